# baseline (speedup 1.0000x reference)
import functools

import jax
import jax.numpy as jnp
from jax import lax
from jax.experimental import pallas as pl
from jax.experimental.pallas import tpu as pltpu
from jax.experimental.pallas import tpu_sc as plsc

NC = 2
NS = 16
NW = NC * NS

B = 16384
F = 26
DIM = 32
BLK = B // NW
QI = 128
NQ = BLK // QI
LANES = 16
NBUF = 3


def _fire_gathers(table_hbm, cidx_v, rows_v, gsem, f, buf):
    for q in range(NQ):
        pltpu.async_copy(
            table_hbm.at[cidx_v.at[f, pl.ds(q * QI, QI)]],
            rows_v.at[buf, pl.ds(q * QI, QI)],
            gsem,
        )


def _wait_gathers(table_hbm, cidx_v, rows_v, gsem, f, buf):
    for q in range(NQ):
        pltpu.make_async_copy(
            table_hbm.at[cidx_v.at[f, pl.ds(q * QI, QI)]],
            rows_v.at[buf, pl.ds(q * QI, QI)],
            gsem,
        ).wait()


def _body(xT_hbm, off_hbm, table_hbm, out_hbm, cidx_v, off_v, rows_v, gsem, ssem):
    wid = lax.axis_index("s") * NC + lax.axis_index("c")
    b0 = wid * BLK

    pltpu.sync_copy(xT_hbm.at[:, pl.ds(b0, BLK)], cidx_v)
    pltpu.sync_copy(off_hbm, off_v)

    def add_f(f, carry):
        off_row = off_v[f, :]

        def add_t(t, c2):
            sl = pl.ds(t * LANES, LANES)
            r = cidx_v[f, sl] + off_row
            cidx_v[f, sl] = ((r & (QROWS - 1)) << 2) + (r >> 18)
            return c2

        return lax.fori_loop(0, BLK // LANES, add_t, carry)

    lax.fori_loop(0, F, add_f, 0)

    for f in range(2):
        _fire_gathers(table_hbm, cidx_v, rows_v, gsem, f, f)

    def chunk_step(j, carry):
        cb = lax.rem(j, NBUF)

        @pl.when(j + 2 < F)
        def _stage_ahead():
            nb = lax.rem(j + 2, NBUF)

            @pl.when(j >= 1)
            def _drain_store():
                pltpu.make_async_copy(
                    rows_v.at[0], out_hbm.at[0, pl.ds(b0, BLK)], ssem
                ).wait()

            _fire_gathers(table_hbm, cidx_v, rows_v, gsem, j + 2, nb)

        _wait_gathers(table_hbm, cidx_v, rows_v, gsem, j, cb)
        pltpu.async_copy(rows_v.at[cb], out_hbm.at[j, pl.ds(b0, BLK)], ssem)
        return carry

    lax.fori_loop(0, F, chunk_step, 0)

    for _ in range(NBUF):
        pltpu.make_async_copy(
            rows_v.at[0], out_hbm.at[0, pl.ds(b0, BLK)], ssem
        ).wait()


TROWS = 1048576
QROWS = TROWS // 4
TBLK = 4096
TGRID = QROWS // TBLK


def _untile_body(x0_ref, x1_ref, x2_ref, x3_ref, o_ref):
    o_ref[...] = jnp.concatenate(
        [x0_ref[...], x1_ref[...], x2_ref[...], x3_ref[...]], axis=1
    )


MAXBLK = (1040002 - 1) // TBLK


def _quarter_spec(k):
    return pl.BlockSpec(
        (TBLK, DIM), lambda i, k=k: (jnp.minimum(k * TGRID + i, MAXBLK), 0)
    )


def _untile(table):
    return pl.pallas_call(
        _untile_body,
        grid=(TGRID,),
        in_specs=[_quarter_spec(k) for k in range(4)],
        out_specs=pl.BlockSpec((TBLK, 128), lambda i: (i, 0)),
        out_shape=jax.ShapeDtypeStruct((QROWS, 128), jnp.float32),
    )(table, table, table, table)


@jax.jit
def _run(xT, off_bcast, table):
    mesh = plsc.VectorSubcoreMesh(
        core_axis_name="c", subcore_axis_name="s", num_cores=NC, num_subcores=NS
    )
    table_lin = _untile(table).reshape(TROWS, DIM)
    fn = pl.kernel(
        _body,
        out_type=jax.ShapeDtypeStruct((F, B, DIM), jnp.float32),
        mesh=mesh,
        scratch_types=[
            pltpu.VMEM((F, BLK), jnp.int32),
            pltpu.VMEM((F, LANES), jnp.int32),
            pltpu.VMEM((NBUF, BLK, DIM), jnp.float32),
            pltpu.SemaphoreType.DMA,
            pltpu.SemaphoreType.DMA,
        ],
        compiler_params=pltpu.CompilerParams(use_tc_tiling_on_sc=False),
    )
    return fn(xT, off_bcast, table_lin)


def kernel(x_categ, table, offsets):
    xT = x_categ.astype(jnp.int32).T
    off_bcast = jnp.broadcast_to(
        offsets.astype(jnp.int32)[:, None], (F, LANES)
    )
    out2 = _run(xT, off_bcast, table)
    return out2.transpose(1, 0, 2)


# speedup vs baseline: 1.4857x; 1.4857x over previous
import functools

import jax
import jax.numpy as jnp
from jax import lax
from jax.experimental import pallas as pl
from jax.experimental.pallas import tpu as pltpu
from jax.experimental.pallas import tpu_sc as plsc

NC = 2
NS = 16
NW = NC * NS

B = 16384
F = 26
DIM = 32
BLK = B // NW
QI = 128
NQ = BLK // QI
LANES = 16
NBUF = 3


def _fire_gathers(table_hbm, cidx_v, rows_v, gsem, f, buf):
    for q in range(NQ):
        pltpu.async_copy(
            table_hbm.at[cidx_v.at[f, pl.ds(q * QI, QI)]],
            rows_v.at[buf, pl.ds(q * QI, QI)],
            gsem,
        )


def _wait_gathers(table_hbm, cidx_v, rows_v, gsem, f, buf):
    for q in range(NQ):
        pltpu.make_async_copy(
            table_hbm.at[cidx_v.at[f, pl.ds(q * QI, QI)]],
            rows_v.at[buf, pl.ds(q * QI, QI)],
            gsem,
        ).wait()


def _body(xT_hbm, off_hbm, table_hbm, out_hbm, cidx_v, off_v, rows_v, gsem, ssem):
    wid = lax.axis_index("s") * NC + lax.axis_index("c")
    b0 = wid * BLK

    pltpu.sync_copy(xT_hbm.at[:, pl.ds(b0, BLK)], cidx_v)
    pltpu.sync_copy(off_hbm, off_v)

    def add_f(f, carry):
        off_row = off_v[f, :]

        def add_t(t, c2):
            sl = pl.ds(t * LANES, LANES)
            r = cidx_v[f, sl] + off_row
            cidx_v[f, sl] = ((r & (QROWS - 1)) << 2) + (r >> 18)
            return c2

        return lax.fori_loop(0, BLK // LANES, add_t, carry)

    lax.fori_loop(0, F, add_f, 0)

    for f in range(2):
        _fire_gathers(table_hbm, cidx_v, rows_v, gsem, f, f)

    def chunk_step(j, carry):
        cb = lax.rem(j, NBUF)

        @pl.when(j + 2 < F)
        def _stage_ahead():
            nb = lax.rem(j + 2, NBUF)

            @pl.when(j >= 1)
            def _drain_store():
                pltpu.make_async_copy(
                    rows_v.at[0], out_hbm.at[0, pl.ds(b0, BLK)], ssem
                ).wait()

            _fire_gathers(table_hbm, cidx_v, rows_v, gsem, j + 2, nb)

        _wait_gathers(table_hbm, cidx_v, rows_v, gsem, j, cb)
        pltpu.async_copy(rows_v.at[cb], out_hbm.at[j, pl.ds(b0, BLK)], ssem)
        return carry

    lax.fori_loop(0, F, chunk_step, 0)

    for _ in range(NBUF):
        pltpu.make_async_copy(
            rows_v.at[0], out_hbm.at[0, pl.ds(b0, BLK)], ssem
        ).wait()


TROWS = 1048576
QROWS = TROWS // 4
TBLK = 4096
TGRID = QROWS // TBLK


def _untile_body(x0_ref, x1_ref, x2_ref, x3_ref, o_ref):
    o_ref[...] = jnp.concatenate(
        [jnp.transpose(x0_ref[...]), jnp.transpose(x1_ref[...]),
         jnp.transpose(x2_ref[...]), jnp.transpose(x3_ref[...])],
        axis=1,
    )


MAXBLK = (1040002 - 1) // TBLK


def _quarter_spec(k):
    return pl.BlockSpec(
        (DIM, TBLK), lambda i, k=k: (0, jnp.minimum(k * TGRID + i, MAXBLK))
    )


def _untile(tableT):
    return pl.pallas_call(
        _untile_body,
        grid=(TGRID,),
        in_specs=[_quarter_spec(k) for k in range(4)],
        out_specs=pl.BlockSpec((TBLK, 128), lambda i: (i, 0)),
        out_shape=jax.ShapeDtypeStruct((QROWS, 128), jnp.float32),
    )(tableT, tableT, tableT, tableT)


@jax.jit
def _run(xT, off_bcast, table):
    mesh = plsc.VectorSubcoreMesh(
        core_axis_name="c", subcore_axis_name="s", num_cores=NC, num_subcores=NS
    )
    table_lin = _untile(table.T).reshape(TROWS, DIM)
    fn = pl.kernel(
        _body,
        out_type=jax.ShapeDtypeStruct((F, B, DIM), jnp.float32),
        mesh=mesh,
        scratch_types=[
            pltpu.VMEM((F, BLK), jnp.int32),
            pltpu.VMEM((F, LANES), jnp.int32),
            pltpu.VMEM((NBUF, BLK, DIM), jnp.float32),
            pltpu.SemaphoreType.DMA,
            pltpu.SemaphoreType.DMA,
        ],
        compiler_params=pltpu.CompilerParams(use_tc_tiling_on_sc=False),
    )
    return fn(xT, off_bcast, table_lin)


def kernel(x_categ, table, offsets):
    xT = x_categ.astype(jnp.int32).T
    off_bcast = jnp.broadcast_to(
        offsets.astype(jnp.int32)[:, None], (F, LANES)
    )
    out2 = _run(xT, off_bcast, table)
    return out2.transpose(1, 0, 2)


# speedup vs baseline: 1.4990x; 1.0089x over previous
import functools

import jax
import jax.numpy as jnp
from jax import lax
from jax.experimental import pallas as pl
from jax.experimental.pallas import tpu as pltpu
from jax.experimental.pallas import tpu_sc as plsc

NC = 2
NS = 16
NW = NC * NS

B = 16384
F = 26
DIM = 32
BLK = B // NW
QI = 128
NQ = BLK // QI
LANES = 16
NBUF = 3


def _fire_gathers(table_hbm, cidx_v, rows_v, gsem, f, buf):
    for q in range(NQ):
        pltpu.async_copy(
            table_hbm.at[cidx_v.at[f, pl.ds(q * QI, QI)]],
            rows_v.at[buf, pl.ds(q * QI, QI)],
            gsem,
        )


def _wait_gathers(table_hbm, cidx_v, rows_v, gsem, f, buf):
    for q in range(NQ):
        pltpu.make_async_copy(
            table_hbm.at[cidx_v.at[f, pl.ds(q * QI, QI)]],
            rows_v.at[buf, pl.ds(q * QI, QI)],
            gsem,
        ).wait()


def _body(xT_hbm, off_hbm, table_hbm, out_hbm, cidx_v, off_v, rows_v, gsem, ssem):
    wid = lax.axis_index("s") * NC + lax.axis_index("c")
    b0 = wid * BLK

    pltpu.sync_copy(xT_hbm.at[:, pl.ds(b0, BLK)], cidx_v)
    pltpu.sync_copy(off_hbm, off_v)

    def add_f(f, carry):
        off_row = off_v[f, :]

        def add_t(t, c2):
            sl = pl.ds(t * LANES, LANES)
            r = cidx_v[f, sl] + off_row
            cidx_v[f, sl] = ((r & (QROWS - 1)) << 2) + (r >> 18)
            return c2

        return lax.fori_loop(0, BLK // LANES, add_t, carry)

    lax.fori_loop(0, F, add_f, 0)

    for f in range(2):
        _fire_gathers(table_hbm, cidx_v, rows_v, gsem, f, f)

    def chunk_step(j, carry):
        cb = lax.rem(j, NBUF)

        @pl.when(j + 2 < F)
        def _stage_ahead():
            nb = lax.rem(j + 2, NBUF)

            @pl.when(j >= 1)
            def _drain_store():
                pltpu.make_async_copy(
                    rows_v.at[0], out_hbm.at[0, pl.ds(b0, BLK)], ssem
                ).wait()

            _fire_gathers(table_hbm, cidx_v, rows_v, gsem, j + 2, nb)

        _wait_gathers(table_hbm, cidx_v, rows_v, gsem, j, cb)
        pltpu.async_copy(rows_v.at[cb], out_hbm.at[j, pl.ds(b0, BLK)], ssem)
        return carry

    lax.fori_loop(0, F, chunk_step, 0)

    for _ in range(NBUF):
        pltpu.make_async_copy(
            rows_v.at[0], out_hbm.at[0, pl.ds(b0, BLK)], ssem
        ).wait()


TROWS = 1048576
QROWS = TROWS // 4
TBLK = 8192
TGRID = QROWS // TBLK


def _untile_body(x0_ref, x1_ref, x2_ref, x3_ref, o_ref):
    o_ref[...] = jnp.concatenate(
        [jnp.transpose(x0_ref[...]), jnp.transpose(x1_ref[...]),
         jnp.transpose(x2_ref[...]), jnp.transpose(x3_ref[...])],
        axis=1,
    )


MAXBLK = (1040002 - 1) // TBLK


def _quarter_spec(k):
    return pl.BlockSpec(
        (DIM, TBLK), lambda i, k=k: (0, jnp.minimum(k * TGRID + i, MAXBLK))
    )


def _untile(tableT):
    return pl.pallas_call(
        _untile_body,
        grid=(TGRID,),
        in_specs=[_quarter_spec(k) for k in range(4)],
        out_specs=pl.BlockSpec((TBLK, 128), lambda i: (i, 0)),
        out_shape=jax.ShapeDtypeStruct((QROWS, 128), jnp.float32),
    )(tableT, tableT, tableT, tableT)


@jax.jit
def _run(xT, off_bcast, table):
    mesh = plsc.VectorSubcoreMesh(
        core_axis_name="c", subcore_axis_name="s", num_cores=NC, num_subcores=NS
    )
    table_lin = _untile(table.T).reshape(TROWS, DIM)
    fn = pl.kernel(
        _body,
        out_type=jax.ShapeDtypeStruct((F, B, DIM), jnp.float32),
        mesh=mesh,
        scratch_types=[
            pltpu.VMEM((F, BLK), jnp.int32),
            pltpu.VMEM((F, LANES), jnp.int32),
            pltpu.VMEM((NBUF, BLK, DIM), jnp.float32),
            pltpu.SemaphoreType.DMA,
            pltpu.SemaphoreType.DMA,
        ],
        compiler_params=pltpu.CompilerParams(use_tc_tiling_on_sc=False),
    )
    return fn(xT, off_bcast, table_lin)


def kernel(x_categ, table, offsets):
    xT = x_categ.astype(jnp.int32).T
    off_bcast = jnp.broadcast_to(
        offsets.astype(jnp.int32)[:, None], (F, LANES)
    )
    out2 = _run(xT, off_bcast, table)
    return out2.transpose(1, 0, 2)
